# baseline (speedup 1.0000x reference)
import functools

import jax
import jax.numpy as jnp
from jax import lax
from jax.experimental import pallas as pl
from jax.experimental.pallas import tpu as pltpu
from jax.experimental.pallas import tpu_sc as plsc

B, Q, V, NY, D = 4096, 200, 100000, 14, 16
NW = 32
BPW = B // NW


def _prep_body(tabT, w, b, g, lb, yt, qt, al, be, lntab_o, vtab_o, qe_o):
    x = tabT[...]
    cid = (lax.broadcasted_iota(jnp.int32, x.shape, 1)
           + pl.program_id(0) * _VBLK)
    x = jnp.where(cid == 0, b[...], jnp.where(cid == 1, w[...] + b[...], x))
    mu = jnp.mean(x, axis=0, keepdims=True)
    var = jnp.mean((x - mu) ** 2, axis=0, keepdims=True)
    y = (x - mu) * lax.rsqrt(var + 1e-5) * g[...]
    lntab_o[...] = jnp.transpose(y)
    vtab_o[...] = yt[...] * al[0, 0]
    qe_o[...] = jnp.transpose(qt[...] * be[0, 0] + lb[...])


VP = 102400
_VBLK = 4096
_fixed = lambda i: (0, 0)
_prep = pl.pallas_call(
    _prep_body,
    grid=(VP // _VBLK,),
    in_specs=[
        pl.BlockSpec((D, _VBLK), lambda i: (0, i)),
        pl.BlockSpec((D, 1), _fixed), pl.BlockSpec((D, 1), _fixed),
        pl.BlockSpec((D, 1), _fixed), pl.BlockSpec((D, 1), _fixed),
        pl.BlockSpec((NY, D), _fixed), pl.BlockSpec((D, Q), _fixed),
        pl.BlockSpec((1, 1), _fixed), pl.BlockSpec((1, 1), _fixed),
    ],
    out_specs=(
        pl.BlockSpec((_VBLK, D), lambda i: (i, 0)),
        pl.BlockSpec((NY, D), _fixed),
        pl.BlockSpec((Q, D), _fixed),
    ),
    out_shape=(
        jax.ShapeDtypeStruct((VP, D), jnp.float32),
        jax.ShapeDtypeStruct((NY, D), jnp.float32),
        jax.ShapeDtypeStruct((Q, D), jnp.float32),
    ),
)


_NB = 4


def _sc_body(lntab, ansT, year, vtab, qe, out, idx_v, year_v, v_v, qe_v,
             vt_v, vt_sh, rbufs, obufs, gsems, osems, psems, vsem):
    sid = lax.axis_index("s")
    wid = sid * 2 + lax.axis_index("c")
    bbase = wid * BPW
    pltpu.sync_copy(ansT.at[:, pl.ds(bbase, BPW)], idx_v)
    pltpu.sync_copy(year.at[pl.ds(bbase, BPW)], year_v)
    pltpu.sync_copy(qe, qe_v)
    pltpu.async_copy(vtab.at[year_v], v_v, vsem).wait()

    lane = lax.broadcasted_iota(jnp.int32, (16,), 0)
    zerov = lane * 0
    cols_d = [jnp.full((16,), d, jnp.int32) for d in range(D)]

    for d in range(D):
        for c in range(BPW // 16):
            vt_v[pl.ds(d * BPW + c * 16, 16)] = plsc.load_gather(
                v_v, [lane + c * 16, cols_d[d]])

    def g_start(s, buf, sem):
        pltpu.make_async_copy(
            lntab.at[idx_v.at[2 * s]], buf.at[pl.ds(0, BPW)], sem).start()
        pltpu.make_async_copy(
            lntab.at[idx_v.at[2 * s + 1]], buf.at[pl.ds(BPW, BPW)],
            sem).start()

    def g_wait(buf, sem):
        pltpu.make_async_copy(
            lntab.at[pl.ds(0, 2 * BPW)], buf, sem).wait()

    def o_start(s, buf, sem):
        for j in range(2):
            for dh in range(2):
                pltpu.make_async_copy(
                    buf.at[pl.ds(j * 2048 + dh * 1024, 1024)],
                    out.at[pl.ds(
                        (((2 * s + j) * 2 + dh) * NW + wid) * 1024, 1024)],
                    sem).start()

    def o_wait(buf, sem):
        pltpu.make_async_copy(buf, out.at[pl.ds(0, 4096)], sem).wait()

    for j in range(2):
        pltpu.sync_copy(vt_v, vt_sh.at[sid, pl.ds(j * 2048, 2048)])

    def p_start(buf, sem):
        pltpu.make_async_copy(vt_sh.at[sid], buf, sem).start()

    def p_wait(buf, sem):
        pltpu.make_async_copy(vt_sh.at[sid], buf, sem).wait()

    NS = Q // 2

    for s0 in range(_NB - 1):
        g_start(s0, rbufs[s0], gsems[s0])
    p_start(obufs[0], psems[0])

    def body(i, carry):
        for k in range(_NB):
            s = i * _NB + k
            kn = (k + 1) % _NB
            rcur = rbufs[k]
            ocur = obufs[k]

            @pl.when(s + _NB - 1 < NS)
            def _():
                g_start(s + _NB - 1, rbufs[(k + _NB - 1) % _NB],
                        gsems[(k + _NB - 1) % _NB])

            @pl.when(s + 1 < NS)
            def _():
                @pl.when(s >= 3)
                def _():
                    o_wait(obufs[kn], osems[kn])

                p_start(obufs[kn], psems[kn])

            g_wait(rcur, gsems[k])
            p_wait(ocur, psems[k])
            q2 = 2 * s

            @plsc.parallel_loop(0, 2 * 8 * D, 1, unroll=32)
            def _(jj):
                jq = jj >> 7
                d = (jj >> 3) & 15
                c = jj & 7
                rows = lane + ((jq << 7) + (c << 4))
                cold = zerov + d
                qidx = zerov + (q2 + jq)
                tv = plsc.load_gather(rcur, [rows, cold])
                qv = plsc.load_gather(qe_v, [qidx, cold])
                plsc.addupdate(ocur.at[pl.ds(jj * 16, 16)], tv + qv)
            o_start(s, ocur, osems[k])
        return carry

    lax.fori_loop(0, NS // _NB, body, 0)
    for k in range(_NB):
        o_wait(obufs[k], osems[k])


_sc_main = functools.partial(
    pl.kernel,
    out_type=jax.ShapeDtypeStruct((Q * 2 * NW * 8 * BPW,), jnp.float32),
    mesh=plsc.VectorSubcoreMesh(core_axis_name="c", subcore_axis_name="s"),
    scratch_types=[
        pltpu.VMEM((Q, BPW), jnp.int32),
        pltpu.VMEM((BPW,), jnp.int32),
        pltpu.VMEM((BPW, D), jnp.float32),
        pltpu.VMEM((Q, D), jnp.float32),
        pltpu.VMEM((D * BPW,), jnp.float32),
        pltpu.VMEM_SHARED((16, 2 * D * BPW), jnp.float32),
        [pltpu.VMEM((2 * BPW, D), jnp.float32)] * _NB,
        [pltpu.VMEM((2 * D * BPW,), jnp.float32)] * _NB,
        [pltpu.SemaphoreType.DMA] * _NB,
        [pltpu.SemaphoreType.DMA] * _NB,
        [pltpu.SemaphoreType.DMA] * _NB,
        pltpu.SemaphoreType.DMA,
    ],
    compiler_params=pltpu.CompilerParams(use_tc_tiling_on_sc=False,
                                         needs_layout_passes=False),
)(_sc_body)


def kernel(year, answer, answer_table, w_cont, b_cont, ln_gamma, ln_beta,
           yearly_table, question_table, alpha, beta):
    year = year.astype(jnp.int32)
    ansT = answer.T.astype(jnp.int32)
    tabT = jnp.pad(answer_table.T, ((0, 0), (0, VP - V)))
    qtT = question_table.T
    lntab, vtab, qe = _prep(
        tabT,
        w_cont.reshape(D, 1), b_cont.reshape(D, 1),
        ln_gamma.reshape(D, 1), ln_beta.reshape(D, 1),
        yearly_table, qtT,
        alpha.reshape(1, 1), beta.reshape(1, 1),
    )
    out5 = _sc_main(lntab, ansT, year, vtab, qe).reshape(Q, 2, NW, 8, BPW)
    return out5.transpose(2, 4, 0, 1, 3).reshape(B, Q, D)


# speedup vs baseline: 1.0064x; 1.0064x over previous
import functools

import jax
import jax.numpy as jnp
from jax import lax
from jax.experimental import pallas as pl
from jax.experimental.pallas import tpu as pltpu
from jax.experimental.pallas import tpu_sc as plsc

B, Q, V, NY, D = 4096, 200, 100000, 14, 16
NW = 32
BPW = B // NW


def _prep_body(tabT, w, b, g, lb, yt, qt, al, be, lntab_o, vtab_o, qe_o):
    x = tabT[...]
    cid = (lax.broadcasted_iota(jnp.int32, x.shape, 1)
           + pl.program_id(0) * _VBLK)
    x = jnp.where(cid == 0, b[...], jnp.where(cid == 1, w[...] + b[...], x))
    mu = jnp.mean(x, axis=0, keepdims=True)
    var = jnp.mean((x - mu) ** 2, axis=0, keepdims=True)
    y = (x - mu) * lax.rsqrt(var + 1e-5) * g[...]
    lntab_o[...] = jnp.transpose(y)
    vtab_o[...] = yt[...] * al[0, 0]
    qe_o[...] = jnp.transpose(qt[...] * be[0, 0] + lb[...])


VP = 102400
_VBLK = 4096
_fixed = lambda i: (0, 0)
_prep = pl.pallas_call(
    _prep_body,
    grid=(VP // _VBLK,),
    in_specs=[
        pl.BlockSpec((D, _VBLK), lambda i: (0, i)),
        pl.BlockSpec((D, 1), _fixed), pl.BlockSpec((D, 1), _fixed),
        pl.BlockSpec((D, 1), _fixed), pl.BlockSpec((D, 1), _fixed),
        pl.BlockSpec((NY, D), _fixed), pl.BlockSpec((D, Q), _fixed),
        pl.BlockSpec((1, 1), _fixed), pl.BlockSpec((1, 1), _fixed),
    ],
    out_specs=(
        pl.BlockSpec((_VBLK, D), lambda i: (i, 0)),
        pl.BlockSpec((NY, D), _fixed),
        pl.BlockSpec((Q, D), _fixed),
    ),
    out_shape=(
        jax.ShapeDtypeStruct((VP, D), jnp.float32),
        jax.ShapeDtypeStruct((NY, D), jnp.float32),
        jax.ShapeDtypeStruct((Q, D), jnp.float32),
    ),
)


_NB = 5


def _sc_body(lntab, ansT, year, vtab, qe, out, idx_v, year_v, v_v, qe_v,
             vt_v, vt_sh, rbufs, obufs, gsems, osems, psems, vsem):
    sid = lax.axis_index("s")
    wid = sid * 2 + lax.axis_index("c")
    bbase = wid * BPW
    pltpu.sync_copy(ansT.at[:, pl.ds(bbase, BPW)], idx_v)
    pltpu.sync_copy(year.at[pl.ds(bbase, BPW)], year_v)
    pltpu.sync_copy(qe, qe_v)
    pltpu.async_copy(vtab.at[year_v], v_v, vsem).wait()

    lane = lax.broadcasted_iota(jnp.int32, (16,), 0)
    zerov = lane * 0
    cols_d = [jnp.full((16,), d, jnp.int32) for d in range(D)]

    for d in range(D):
        for c in range(BPW // 16):
            vt_v[pl.ds(d * BPW + c * 16, 16)] = plsc.load_gather(
                v_v, [lane + c * 16, cols_d[d]])

    def g_start(s, buf, sem):
        pltpu.make_async_copy(
            lntab.at[idx_v.at[2 * s]], buf.at[pl.ds(0, BPW)], sem).start()
        pltpu.make_async_copy(
            lntab.at[idx_v.at[2 * s + 1]], buf.at[pl.ds(BPW, BPW)],
            sem).start()

    def g_wait(buf, sem):
        pltpu.make_async_copy(
            lntab.at[pl.ds(0, 2 * BPW)], buf, sem).wait()

    def o_start(s, buf, sem):
        for j in range(2):
            for dh in range(2):
                pltpu.make_async_copy(
                    buf.at[pl.ds(j * 2048 + dh * 1024, 1024)],
                    out.at[pl.ds(
                        (((2 * s + j) * 2 + dh) * NW + wid) * 1024, 1024)],
                    sem).start()

    def o_wait(buf, sem):
        pltpu.make_async_copy(buf, out.at[pl.ds(0, 4096)], sem).wait()

    for j in range(2):
        pltpu.sync_copy(vt_v, vt_sh.at[sid, pl.ds(j * 2048, 2048)])

    def p_start(buf, sem):
        pltpu.make_async_copy(vt_sh.at[sid], buf, sem).start()

    def p_wait(buf, sem):
        pltpu.make_async_copy(vt_sh.at[sid], buf, sem).wait()

    NS = Q // 2

    for s0 in range(_NB - 1):
        g_start(s0, rbufs[s0], gsems[s0])
    p_start(obufs[0], psems[0])

    def body(i, carry):
        for k in range(_NB):
            s = i * _NB + k
            kn = (k + 1) % _NB
            rcur = rbufs[k]
            ocur = obufs[k]

            @pl.when(s + _NB - 1 < NS)
            def _():
                g_start(s + _NB - 1, rbufs[(k + _NB - 1) % _NB],
                        gsems[(k + _NB - 1) % _NB])

            @pl.when(s + 1 < NS)
            def _():
                @pl.when(s >= _NB - 1)
                def _():
                    o_wait(obufs[kn], osems[kn])

                p_start(obufs[kn], psems[kn])

            g_wait(rcur, gsems[k])
            p_wait(ocur, psems[k])
            q2 = 2 * s

            @plsc.parallel_loop(0, 2 * 8 * D, 1, unroll=16)
            def _(jj):
                jq = jj >> 7
                d = (jj >> 3) & 15
                c = jj & 7
                rows = lane + ((jq << 7) + (c << 4))
                cold = zerov + d
                qidx = zerov + (q2 + jq)
                tv = plsc.load_gather(rcur, [rows, cold])
                qv = plsc.load_gather(qe_v, [qidx, cold])
                plsc.addupdate(ocur.at[pl.ds(jj * 16, 16)], tv + qv)
            o_start(s, ocur, osems[k])
        return carry

    lax.fori_loop(0, NS // _NB, body, 0)
    for k in range(_NB):
        o_wait(obufs[k], osems[k])


_sc_main = functools.partial(
    pl.kernel,
    out_type=jax.ShapeDtypeStruct((Q * 2 * NW * 8 * BPW,), jnp.float32),
    mesh=plsc.VectorSubcoreMesh(core_axis_name="c", subcore_axis_name="s"),
    scratch_types=[
        pltpu.VMEM((Q, BPW), jnp.int32),
        pltpu.VMEM((BPW,), jnp.int32),
        pltpu.VMEM((BPW, D), jnp.float32),
        pltpu.VMEM((Q, D), jnp.float32),
        pltpu.VMEM((D * BPW,), jnp.float32),
        pltpu.VMEM_SHARED((16, 2 * D * BPW), jnp.float32),
        [pltpu.VMEM((2 * BPW, D), jnp.float32)] * _NB,
        [pltpu.VMEM((2 * D * BPW,), jnp.float32)] * _NB,
        [pltpu.SemaphoreType.DMA] * _NB,
        [pltpu.SemaphoreType.DMA] * _NB,
        [pltpu.SemaphoreType.DMA] * _NB,
        pltpu.SemaphoreType.DMA,
    ],
    compiler_params=pltpu.CompilerParams(use_tc_tiling_on_sc=False,
                                         needs_layout_passes=False),
)(_sc_body)


def kernel(year, answer, answer_table, w_cont, b_cont, ln_gamma, ln_beta,
           yearly_table, question_table, alpha, beta):
    year = year.astype(jnp.int32)
    ansT = answer.T.astype(jnp.int32)
    tabT = jnp.pad(answer_table.T, ((0, 0), (0, VP - V)))
    qtT = question_table.T
    lntab, vtab, qe = _prep(
        tabT,
        w_cont.reshape(D, 1), b_cont.reshape(D, 1),
        ln_gamma.reshape(D, 1), ln_beta.reshape(D, 1),
        yearly_table, qtT,
        alpha.reshape(1, 1), beta.reshape(1, 1),
    )
    out5 = _sc_main(lntab, ansT, year, vtab, qe).reshape(Q, 2, NW, 8, BPW)
    return out5.transpose(2, 4, 0, 1, 3).reshape(B, Q, D)


# speedup vs baseline: 1.0350x; 1.0284x over previous
import functools

import jax
import jax.numpy as jnp
from jax import lax
from jax.experimental import pallas as pl
from jax.experimental.pallas import tpu as pltpu
from jax.experimental.pallas import tpu_sc as plsc

B, Q, V, NY, D = 4096, 200, 100000, 14, 16
NW = 32
BPW = B // NW


def _prep_body(tabT, w, b, g, lb, yt, qt, al, be, lntab_o, vtab_o, qe_o):
    x = tabT[...]
    cid = (lax.broadcasted_iota(jnp.int32, x.shape, 1)
           + pl.program_id(0) * _VBLK)
    x = jnp.where(cid == 0, b[...], jnp.where(cid == 1, w[...] + b[...], x))
    mu = jnp.mean(x, axis=0, keepdims=True)
    var = jnp.mean((x - mu) ** 2, axis=0, keepdims=True)
    y = (x - mu) * lax.rsqrt(var + 1e-5) * g[...]
    lntab_o[...] = jnp.transpose(y)
    vtab_o[...] = yt[...] * al[0, 0]
    qe_o[...] = jnp.transpose(qt[...] * be[0, 0] + lb[...])


VP = 102400
_VBLK = 4096
_fixed = lambda i: (0, 0)
_prep = pl.pallas_call(
    _prep_body,
    grid=(VP // _VBLK,),
    in_specs=[
        pl.BlockSpec((D, _VBLK), lambda i: (0, i)),
        pl.BlockSpec((D, 1), _fixed), pl.BlockSpec((D, 1), _fixed),
        pl.BlockSpec((D, 1), _fixed), pl.BlockSpec((D, 1), _fixed),
        pl.BlockSpec((NY, D), _fixed), pl.BlockSpec((D, Q), _fixed),
        pl.BlockSpec((1, 1), _fixed), pl.BlockSpec((1, 1), _fixed),
    ],
    out_specs=(
        pl.BlockSpec((_VBLK, D), lambda i: (i, 0)),
        pl.BlockSpec((NY, D), _fixed),
        pl.BlockSpec((Q, D), _fixed),
    ),
    out_shape=(
        jax.ShapeDtypeStruct((VP, D), jnp.float32),
        jax.ShapeDtypeStruct((NY, D), jnp.float32),
        jax.ShapeDtypeStruct((Q, D), jnp.float32),
    ),
)


_NB = 4


def _sc_body(lntab, ansT, year, vtab, qe, out, idx_v, year_v, v_v, qe_v,
             vt_v, vt_sh, rbufs, obufs, gsems, osems, psems, vsem):
    sid = lax.axis_index("s")
    wid = sid * 2 + lax.axis_index("c")
    bbase = wid * BPW
    pltpu.sync_copy(ansT.at[:, pl.ds(bbase, BPW)], idx_v)
    pltpu.sync_copy(year.at[pl.ds(bbase, BPW)], year_v)
    pltpu.sync_copy(qe, qe_v)
    pltpu.async_copy(vtab.at[year_v], v_v, vsem).wait()

    lane = lax.broadcasted_iota(jnp.int32, (16,), 0)
    zerov = lane * 0
    cols_d = [jnp.full((16,), d, jnp.int32) for d in range(D)]

    for d in range(D):
        for c in range(BPW // 16):
            vt_v[pl.ds(d * BPW + c * 16, 16)] = plsc.load_gather(
                v_v, [lane + c * 16, cols_d[d]])

    def g_start(s, buf, sem):
        pltpu.make_async_copy(
            lntab.at[idx_v.at[2 * s]], buf.at[pl.ds(0, BPW)], sem).start()
        pltpu.make_async_copy(
            lntab.at[idx_v.at[2 * s + 1]], buf.at[pl.ds(BPW, BPW)],
            sem).start()

    def g_wait(buf, sem):
        pltpu.make_async_copy(
            lntab.at[pl.ds(0, 2 * BPW)], buf, sem).wait()

    def o_start(s, buf, sem):
        for j in range(2):
            for dh in range(2):
                pltpu.make_async_copy(
                    buf.at[pl.ds(j * 2048 + dh * 1024, 1024)],
                    out.at[pl.ds(
                        (((2 * s + j) * 2 + dh) * NW + wid) * 1024, 1024)],
                    sem).start()

    def o_wait(buf, sem):
        pltpu.make_async_copy(buf, out.at[pl.ds(0, 4096)], sem).wait()

    for j in range(2):
        pltpu.sync_copy(vt_v, vt_sh.at[sid, pl.ds(j * 2048, 2048)])

    def p_start(buf, sem):
        pltpu.make_async_copy(vt_sh.at[sid], buf, sem).start()

    def p_wait(buf, sem):
        pltpu.make_async_copy(vt_sh.at[sid], buf, sem).wait()

    NS = Q // 2

    for s0 in range(_NB - 1):
        g_start(s0, rbufs[s0], gsems[s0])

    def body(i, carry):
        for k in range(_NB):
            s = i * _NB + k
            kn = (k + 1) % _NB
            rcur = rbufs[k]
            ocur = obufs[k]

            @pl.when(s + _NB - 1 < NS)
            def _():
                g_start(s + _NB - 1, rbufs[(k + _NB - 1) % _NB],
                        gsems[(k + _NB - 1) % _NB])

            @pl.when(s >= _NB)
            def _():
                o_wait(ocur, osems[k])

            g_wait(rcur, gsems[k])
            q2 = 2 * s

            @plsc.parallel_loop(0, 2 * 8 * D, 1, unroll=16)
            def _(jj):
                jq = jj >> 7
                d = (jj >> 3) & 15
                c = jj & 7
                rows = lane + ((jq << 7) + (c << 4))
                cold = zerov + d
                qidx = zerov + (q2 + jq)
                tv = plsc.load_gather(rcur, [rows, cold])
                qv = plsc.load_gather(qe_v, [qidx, cold])
                vv = vt_v[pl.ds((jj & 127) * 16, 16)]
                ocur[pl.ds(jj * 16, 16)] = tv + qv + vv
            o_start(s, ocur, osems[k])
        return carry

    lax.fori_loop(0, NS // _NB, body, 0)
    for k in range(_NB):
        o_wait(obufs[k], osems[k])


_sc_main = functools.partial(
    pl.kernel,
    out_type=jax.ShapeDtypeStruct((Q * 2 * NW * 8 * BPW,), jnp.float32),
    mesh=plsc.VectorSubcoreMesh(core_axis_name="c", subcore_axis_name="s"),
    scratch_types=[
        pltpu.VMEM((Q, BPW), jnp.int32),
        pltpu.VMEM((BPW,), jnp.int32),
        pltpu.VMEM((BPW, D), jnp.float32),
        pltpu.VMEM((Q, D), jnp.float32),
        pltpu.VMEM((D * BPW,), jnp.float32),
        pltpu.VMEM_SHARED((16, 2 * D * BPW), jnp.float32),
        [pltpu.VMEM((2 * BPW, D), jnp.float32)] * _NB,
        [pltpu.VMEM((2 * D * BPW,), jnp.float32)] * _NB,
        [pltpu.SemaphoreType.DMA] * _NB,
        [pltpu.SemaphoreType.DMA] * _NB,
        [pltpu.SemaphoreType.DMA] * _NB,
        pltpu.SemaphoreType.DMA,
    ],
    compiler_params=pltpu.CompilerParams(use_tc_tiling_on_sc=False,
                                         needs_layout_passes=False),
)(_sc_body)


def kernel(year, answer, answer_table, w_cont, b_cont, ln_gamma, ln_beta,
           yearly_table, question_table, alpha, beta):
    year = year.astype(jnp.int32)
    ansT = answer.T.astype(jnp.int32)
    tabT = jnp.pad(answer_table.T, ((0, 0), (0, VP - V)))
    qtT = question_table.T
    lntab, vtab, qe = _prep(
        tabT,
        w_cont.reshape(D, 1), b_cont.reshape(D, 1),
        ln_gamma.reshape(D, 1), ln_beta.reshape(D, 1),
        yearly_table, qtT,
        alpha.reshape(1, 1), beta.reshape(1, 1),
    )
    out5 = _sc_main(lntab, ansT, year, vtab, qe).reshape(Q, 2, NW, 8, BPW)
    return out5.transpose(2, 4, 0, 1, 3).reshape(B, Q, D)


# speedup vs baseline: 1.0428x; 1.0076x over previous
import functools

import jax
import jax.numpy as jnp
from jax import lax
from jax.experimental import pallas as pl
from jax.experimental.pallas import tpu as pltpu
from jax.experimental.pallas import tpu_sc as plsc

B, Q, V, NY, D = 4096, 200, 100000, 14, 16
NW = 32
BPW = B // NW


def _prep_body(tabT, w, b, g, lb, yt, qt, al, be, lntab_o, vtab_o, qe_o):
    x = tabT[...]
    cid = (lax.broadcasted_iota(jnp.int32, x.shape, 1)
           + pl.program_id(0) * _VBLK)
    x = jnp.where(cid == 0, b[...], jnp.where(cid == 1, w[...] + b[...], x))
    mu = jnp.mean(x, axis=0, keepdims=True)
    var = jnp.mean((x - mu) ** 2, axis=0, keepdims=True)
    y = (x - mu) * lax.rsqrt(var + 1e-5) * g[...]
    lntab_o[...] = jnp.transpose(y)
    vtab_o[...] = yt[...] * al[0, 0]
    qe_o[...] = jnp.transpose(qt[...] * be[0, 0] + lb[...])


VP = 102400
_VBLK = 4096
_fixed = lambda i: (0, 0)
_prep = pl.pallas_call(
    _prep_body,
    grid=(VP // _VBLK,),
    in_specs=[
        pl.BlockSpec((D, _VBLK), lambda i: (0, i)),
        pl.BlockSpec((D, 1), _fixed), pl.BlockSpec((D, 1), _fixed),
        pl.BlockSpec((D, 1), _fixed), pl.BlockSpec((D, 1), _fixed),
        pl.BlockSpec((NY, D), _fixed), pl.BlockSpec((D, Q), _fixed),
        pl.BlockSpec((1, 1), _fixed), pl.BlockSpec((1, 1), _fixed),
    ],
    out_specs=(
        pl.BlockSpec((_VBLK, D), lambda i: (i, 0)),
        pl.BlockSpec((NY, D), _fixed),
        pl.BlockSpec((Q, D), _fixed),
    ),
    out_shape=(
        jax.ShapeDtypeStruct((VP, D), jnp.float32),
        jax.ShapeDtypeStruct((NY, D), jnp.float32),
        jax.ShapeDtypeStruct((Q, D), jnp.float32),
    ),
)


_NB = 4


def _sc_body(lntab, ansT, year, vtab, qe, out, idx_v, year_v, v_v, qe_v,
             vt_v, rbufs, obufs, gsems, osems, vsem):
    wid = lax.axis_index("s") * 2 + lax.axis_index("c")
    bbase = wid * BPW
    pltpu.sync_copy(ansT.at[:, pl.ds(bbase, BPW)], idx_v)
    pltpu.sync_copy(year.at[pl.ds(bbase, BPW)], year_v)
    pltpu.sync_copy(qe, qe_v)
    pltpu.async_copy(vtab.at[year_v], v_v, vsem).wait()

    lane = lax.broadcasted_iota(jnp.int32, (16,), 0)
    zerov = lane * 0
    cols_d = [jnp.full((16,), d, jnp.int32) for d in range(D)]

    for d in range(D):
        for c in range(BPW // 16):
            vt_v[pl.ds(d * BPW + c * 16, 16)] = plsc.load_gather(
                v_v, [lane + c * 16, cols_d[d]])

    def g_start(s, buf, sem):
        pltpu.make_async_copy(
            lntab.at[idx_v.at[2 * s]], buf.at[pl.ds(0, BPW)], sem).start()
        pltpu.make_async_copy(
            lntab.at[idx_v.at[2 * s + 1]], buf.at[pl.ds(BPW, BPW)],
            sem).start()

    def g_wait(buf, sem):
        pltpu.make_async_copy(
            lntab.at[pl.ds(0, 2 * BPW)], buf, sem).wait()

    def o_start(s, buf, sem):
        for j in range(2):
            for dh in range(2):
                pltpu.make_async_copy(
                    buf.at[pl.ds(j * 2048 + dh * 1024, 1024)],
                    out.at[pl.ds(
                        (((2 * s + j) * 2 + dh) * NW + wid) * 1024, 1024)],
                    sem).start()

    def o_wait(buf, sem):
        pltpu.make_async_copy(buf, out.at[pl.ds(0, 4096)], sem).wait()

    NS = Q // 2

    for s0 in range(_NB - 1):
        g_start(s0, rbufs[s0], gsems[s0])

    def body(i, carry):
        for k in range(_NB):
            s = i * _NB + k
            kn = (k + 1) % _NB
            rcur = rbufs[k]
            ocur = obufs[k]

            @pl.when(s + _NB - 1 < NS)
            def _():
                g_start(s + _NB - 1, rbufs[(k + _NB - 1) % _NB],
                        gsems[(k + _NB - 1) % _NB])

            @pl.when(s >= _NB)
            def _():
                o_wait(ocur, osems[k])

            g_wait(rcur, gsems[k])
            q2 = 2 * s

            @plsc.parallel_loop(0, 2 * 8 * D, 1, unroll=16)
            def _(jj):
                jq = jj >> 7
                d = (jj >> 3) & 15
                c = jj & 7
                rows = lane + ((jq << 7) + (c << 4))
                cold = zerov + d
                qidx = zerov + (q2 + jq)
                tv = plsc.load_gather(rcur, [rows, cold])
                qv = plsc.load_gather(qe_v, [qidx, cold])
                vv = vt_v[pl.ds((jj & 127) * 16, 16)]
                ocur[pl.ds(jj * 16, 16)] = tv + qv + vv
            o_start(s, ocur, osems[k])
        return carry

    lax.fori_loop(0, NS // _NB, body, 0)
    for k in range(_NB):
        o_wait(obufs[k], osems[k])


_sc_main = functools.partial(
    pl.kernel,
    out_type=jax.ShapeDtypeStruct((Q * 2 * NW * 8 * BPW,), jnp.float32),
    mesh=plsc.VectorSubcoreMesh(core_axis_name="c", subcore_axis_name="s"),
    scratch_types=[
        pltpu.VMEM((Q, BPW), jnp.int32),
        pltpu.VMEM((BPW,), jnp.int32),
        pltpu.VMEM((BPW, D), jnp.float32),
        pltpu.VMEM((Q, D), jnp.float32),
        pltpu.VMEM((D * BPW,), jnp.float32),
        [pltpu.VMEM((2 * BPW, D), jnp.float32)] * _NB,
        [pltpu.VMEM((2 * D * BPW,), jnp.float32)] * _NB,
        [pltpu.SemaphoreType.DMA] * _NB,
        [pltpu.SemaphoreType.DMA] * _NB,
        pltpu.SemaphoreType.DMA,
    ],
    compiler_params=pltpu.CompilerParams(use_tc_tiling_on_sc=False,
                                         needs_layout_passes=False),
)(_sc_body)


def kernel(year, answer, answer_table, w_cont, b_cont, ln_gamma, ln_beta,
           yearly_table, question_table, alpha, beta):
    year = year.astype(jnp.int32)
    ansT = answer.T.astype(jnp.int32)
    tabT = jnp.pad(answer_table.T, ((0, 0), (0, VP - V)))
    qtT = question_table.T
    lntab, vtab, qe = _prep(
        tabT,
        w_cont.reshape(D, 1), b_cont.reshape(D, 1),
        ln_gamma.reshape(D, 1), ln_beta.reshape(D, 1),
        yearly_table, qtT,
        alpha.reshape(1, 1), beta.reshape(1, 1),
    )
    out5 = _sc_main(lntab, ansT, year, vtab, qe).reshape(Q, 2, NW, 8, BPW)
    return out5.transpose(2, 4, 0, 1, 3).reshape(B, Q, D)
